# baseline (speedup 1.0000x reference)
import functools

import jax
import jax.numpy as jnp
from jax import lax
from jax.experimental import pallas as pl
from jax.experimental.pallas import tpu as pltpu
from jax.experimental.pallas import tpu_sc as plsc

ROWS = 16384
COLS = 64
TOTAL = ROWS * COLS
NC = 2
NS = 16
NW = NC * NS
N_PER = TOTAL // NW
LANES = 16
VECS = N_PER // LANES


def _sc_gather(table_flat, idx_flat):
    mesh = plsc.VectorSubcoreMesh(core_axis_name="c", subcore_axis_name="s")

    @functools.partial(
        pl.kernel,
        out_type=jax.ShapeDtypeStruct((TOTAL,), jnp.float32),
        mesh=mesh,
        scratch_types=[
            pltpu.VMEM((N_PER,), jnp.int32),
            pltpu.VMEM((N_PER,), jnp.float32),
            pltpu.SemaphoreType.DMA,
        ],
    )
    def k(table_hbm, idx_hbm, out_hbm, idx_v, out_v, sem):
        wid = lax.axis_index("s") * NC + lax.axis_index("c")
        base = wid * N_PER

        pltpu.sync_copy(idx_hbm.at[pl.ds(base, N_PER)], idx_v)

        lane = lax.iota(jnp.int32, LANES)

        def body(t, _):
            v = idx_v[pl.ds(t * LANES, LANES)]
            col0 = (t % 4) * LANES
            idx_v[pl.ds(t * LANES, LANES)] = v * COLS + col0 + lane
            return 0

        lax.fori_loop(0, VECS, body, 0, unroll=8)

        pltpu.async_copy(table_hbm.at[idx_v], out_v, sem).wait()

        pltpu.sync_copy(out_v, out_hbm.at[pl.ds(base, N_PER)])

    return k(table_flat, idx_flat)


def kernel(input_tensor, index_tensor):
    table_flat = input_tensor.reshape(-1)
    idx_flat = index_tensor.reshape(-1)
    out_flat = _sc_gather(table_flat, idx_flat)
    return out_flat.reshape(ROWS, COLS)


# speedup vs baseline: 3.3448x; 3.3448x over previous
import functools

import jax
import jax.numpy as jnp
from jax import lax
from jax.experimental import pallas as pl
from jax.experimental.pallas import tpu as pltpu
from jax.experimental.pallas import tpu_sc as plsc

ROWS = 16384
COLS = 64
TABLE_ROWS = 1000000
NC = 2
NS = 16
COLS_PER_SC = COLS // NC
R_PER_TILE = ROWS // NS


def _sc_gather(table_t, idx_t):
    mesh = plsc.VectorSubcoreMesh(core_axis_name="c", subcore_axis_name="s")

    @functools.partial(
        pl.kernel,
        out_type=jax.ShapeDtypeStruct((COLS, ROWS), jnp.float32),
        mesh=mesh,
        scratch_types=[
            pltpu.VMEM_SHARED((TABLE_ROWS,), jnp.float32),
            pltpu.VMEM_SHARED((TABLE_ROWS,), jnp.float32),
            pltpu.VMEM((2 * R_PER_TILE,), jnp.int32),
            pltpu.VMEM((2 * R_PER_TILE,), jnp.float32),
            pltpu.SemaphoreType.DMA,
            pltpu.SemaphoreType.DMA,
            pltpu.SemaphoreType.DMA,
            pltpu.SemaphoreType.DMA,
            pltpu.SemaphoreType.DMA,
            pltpu.SemaphoreType.DMA,
        ],
    )
    def k(tab, idx_hbm, out_hbm, sp_a, sp_b, idx_v, gbuf, isem_a, isem_b,
          ssem, gsem, wsem_a, wsem_b):
        sc = lax.axis_index("c")
        tid = lax.axis_index("s")
        c_base = sc * COLS_PER_SC
        r0 = tid * R_PER_TILE

        def idx_copy(c):
            par = c % 2
            return pltpu.make_async_copy(
                idx_hbm.at[c_base + c, pl.ds(r0, R_PER_TILE)],
                idx_v.at[pl.ds(par * R_PER_TILE, R_PER_TILE)],
                isem_a if par == 0 else isem_b,
            )

        def out_copy(c):
            par = c % 2
            return pltpu.make_async_copy(
                gbuf.at[pl.ds(par * R_PER_TILE, R_PER_TILE)],
                out_hbm.at[c_base + c, pl.ds(r0, R_PER_TILE)],
                wsem_a if par == 0 else wsem_b,
            )

        def stage_copy(c):
            return pltpu.make_async_copy(
                tab.at[c_base + c], sp_a if c % 2 == 0 else sp_b, ssem
            )

        idx_copy(0).start()

        @pl.when(tid == 0)
        def _():
            stage_copy(0).start()

        for c in range(COLS_PER_SC):
            par = c % 2
            buf = sp_a if par == 0 else sp_b
            sl = pl.ds(par * R_PER_TILE, R_PER_TILE)

            if c + 1 < COLS_PER_SC:
                idx_copy(c + 1).start()

            @pl.when(tid == 0)
            def _():
                stage_copy(c).wait()

            plsc.subcore_barrier()

            if c + 1 < COLS_PER_SC:

                @pl.when(tid == 0)
                def _():
                    stage_copy(c + 1).start()

            idx_copy(c).wait()

            if c >= 2:
                out_copy(c - 2).wait()

            pltpu.async_copy(buf.at[idx_v.at[sl]], gbuf.at[sl], gsem).wait()
            out_copy(c).start()

        out_copy(COLS_PER_SC - 2).wait()
        out_copy(COLS_PER_SC - 1).wait()

    return k(table_t, idx_t)


def kernel(input_tensor, index_tensor):
    out_t = _sc_gather(input_tensor.T, index_tensor.T)
    return out_t.T


# speedup vs baseline: 3.8940x; 1.1642x over previous
import functools

import jax
import jax.numpy as jnp
from jax import lax
from jax.experimental import pallas as pl
from jax.experimental.pallas import tpu as pltpu
from jax.experimental.pallas import tpu_sc as plsc

ROWS = 16384
COLS = 64
TABLE_ROWS = 1000000
NC = 2
NS = 16
COLS_PER_SC = COLS // NC
R_PER_TILE = ROWS // NS


def _sc_gather(table_t, idx_t):
    mesh = plsc.VectorSubcoreMesh(core_axis_name="c", subcore_axis_name="s")

    @functools.partial(
        pl.kernel,
        out_type=jax.ShapeDtypeStruct((COLS, ROWS), jnp.float32),
        mesh=mesh,
        scratch_types=[
            pltpu.VMEM_SHARED((TABLE_ROWS,), jnp.float32),
            pltpu.VMEM_SHARED((TABLE_ROWS,), jnp.float32),
            pltpu.VMEM((2 * R_PER_TILE,), jnp.int32),
            pltpu.VMEM((2 * R_PER_TILE,), jnp.float32),
            pltpu.SemaphoreType.DMA,
            pltpu.SemaphoreType.DMA,
            pltpu.SemaphoreType.DMA,
            pltpu.SemaphoreType.DMA,
            pltpu.SemaphoreType.DMA,
            pltpu.SemaphoreType.DMA,
        ],
    )
    def k(tab, idx_hbm, out_hbm, sp_a, sp_b, idx_v, gbuf, isem_a, isem_b,
          ssem, gsem, wsem_a, wsem_b):
        sc = lax.axis_index("c")
        tid = lax.axis_index("s")
        c_base = sc * COLS_PER_SC
        r0 = tid * R_PER_TILE

        def idx_copy(c):
            par = c % 2
            return pltpu.make_async_copy(
                idx_hbm.at[c_base + c, pl.ds(r0, R_PER_TILE)],
                idx_v.at[pl.ds(par * R_PER_TILE, R_PER_TILE)],
                isem_a if par == 0 else isem_b,
            )

        def out_copy(c):
            par = c % 2
            return pltpu.make_async_copy(
                gbuf.at[pl.ds(par * R_PER_TILE, R_PER_TILE)],
                out_hbm.at[c_base + c, pl.ds(r0, R_PER_TILE)],
                wsem_a if par == 0 else wsem_b,
            )

        def stage_copy(c):
            return pltpu.make_async_copy(
                tab.at[c_base + c], sp_a if c % 2 == 0 else sp_b, ssem
            )

        idx_copy(0).start()

        @pl.when(tid == 0)
        def _():
            stage_copy(0).start()

        @pl.when(tid == 1)
        def _():
            stage_copy(1).start()

        for c in range(COLS_PER_SC):
            par = c % 2
            buf = sp_a if par == 0 else sp_b
            sl = pl.ds(par * R_PER_TILE, R_PER_TILE)

            if c + 1 < COLS_PER_SC:
                idx_copy(c + 1).start()

            @pl.when(tid == par)
            def _():
                stage_copy(c).wait()

            plsc.subcore_barrier()

            idx_copy(c).wait()

            if c >= 2:
                out_copy(c - 2).wait()

            pltpu.async_copy(buf.at[idx_v.at[sl]], gbuf.at[sl], gsem).wait()
            out_copy(c).start()

            if c + 2 < COLS_PER_SC:
                plsc.subcore_barrier()

                @pl.when(tid == par)
                def _():
                    stage_copy(c + 2).start()

        out_copy(COLS_PER_SC - 2).wait()
        out_copy(COLS_PER_SC - 1).wait()

    return k(table_t, idx_t)


def kernel(input_tensor, index_tensor):
    out_t = _sc_gather(input_tensor.T, index_tensor.T)
    return out_t.T
